# Initial kernel scaffold; baseline (speedup 1.0000x reference)
import jax
import jax.numpy as jnp
from jax.experimental import pallas as pl


def kernel(query, value, reference_points, Wv, bv, Woff, boff, Watt, batt, Wout, bout):
    raise NotImplementedError("write your pallas kernel here")


# speedup vs baseline: 97.0612x; 97.0612x over previous
import functools

import numpy as np
import jax
import jax.numpy as jnp
from jax import lax
from jax.experimental import pallas as pl
from jax.experimental.pallas import tpu as pltpu
from jax.experimental.pallas import tpu_sc as plsc

_SPATIAL = [(64, 64), (32, 32), (16, 16), (8, 8)]
_NHEADS = 8
_NLEVELS = 4
_NPOINTS = 4
_DMODEL = 256
_HDIM = 32
_NCHAN = _NHEADS * _NLEVELS * _NPOINTS
_OFFS = [0]
for _h, _w in _SPATIAL:
    _OFFS.append(_OFFS[-1] + _h * _w)
_LEN_V = _OFFS[-1]

_c = np.arange(_NCHAN)
_hh = _c // (_NLEVELS * _NPOINTS)
_ll = (_c // _NPOINTS) % _NLEVELS
_Hl = np.array([s[0] for s in _SPATIAL], np.int64)[_ll]
_Wl = np.array([s[1] for s in _SPATIAL], np.int64)[_ll]
_CF = np.stack([
    (1.0 / _Hl.astype(np.float64)).astype(np.float32),
    (1.0 / _Wl.astype(np.float64)).astype(np.float32),
    (_Wl - 2).astype(np.float32),
    (_Hl - 2).astype(np.float32),
])
_CI = np.stack([
    (np.array(_OFFS[:-1], np.int64)[_ll] * _NHEADS).astype(np.int32),
    (_Wl * _NHEADS).astype(np.int32),
    _hh.astype(np.int32),
])
_GMAT = (_c.reshape(_NCHAN, 1) // (_NLEVELS * _NPOINTS) ==
         np.arange(_NHEADS).reshape(1, _NHEADS)).astype(np.float32)
_GMATT = np.ascontiguousarray(_GMAT.T)

_QBLK = 680


def _ab_body(q_ref, val_ref, rpx_ref, rpy_ref, Wv_ref, bv_ref, Wox_ref,
             box_ref, Woy_ref, boy_ref, Wa_ref, ba_ref, cf_ref, ci_ref,
             g_ref, gt_ref, v_ref, idx_ref, w_ref):
    b = pl.program_id(0)
    q = q_ref[0]
    v_ref[0] = jnp.dot(val_ref[0], Wv_ref[...]) + bv_ref[...]
    sox = jnp.dot(q, Wox_ref[...]) + box_ref[...]
    soy = jnp.dot(q, Woy_ref[...]) + boy_ref[...]
    logits = jnp.dot(q, Wa_ref[...]) + ba_ref[...]
    m = jnp.max(logits, axis=-1, keepdims=True)
    e = jnp.exp(logits - m)
    aw1 = e / jnp.sum(e, axis=-1, keepdims=True)
    e2 = jnp.exp(aw1)
    den = jnp.dot(jnp.dot(e2, g_ref[...], precision=lax.Precision.HIGHEST),
                  gt_ref[...], precision=lax.Precision.HIGHEST)
    aw2 = e2 / den
    wm2f = cf_ref[2:3]
    hm2f = cf_ref[3:4]
    cx = rpx_ref[0] + sox * cf_ref[0:1]
    cy = rpy_ref[0] + soy * cf_ref[1:2]
    x = 0.5 * ((cx + 1.0) * wm2f)
    y = 0.5 * ((cy + 1.0) * hm2f)
    x0f = jnp.floor(x)
    y0f = jnp.floor(y)
    fx = x - x0f
    fy = y - y0f
    valid = ((x0f >= 0.0) & (x0f <= wm2f) & (y0f >= 0.0) & (y0f <= hm2f))
    awv = jnp.where(valid, aw2, 0.0)
    x0 = jnp.clip(x0f, 0.0, wm2f).astype(jnp.int32)
    y0 = jnp.clip(y0f, 0.0, hm2f).astype(jnp.int32)
    w8 = ci_ref[1:2]
    base = (b * (_LEN_V * _NHEADS) + ci_ref[0:1]
            + y0 * w8 + x0 * _NHEADS + ci_ref[2:3])
    idx_ref[0, :, 0:128] = base
    idx_ref[0, :, 128:256] = base + _NHEADS
    idx_ref[0, :, 256:384] = base + w8
    idx_ref[0, :, 384:512] = base + w8 + _NHEADS
    gx = 1.0 - fx
    gy = 1.0 - fy
    w_ref[0, :, 0:128] = gx * gy * awv
    w_ref[0, :, 128:256] = fx * gy * awv
    w_ref[0, :, 256:384] = gx * fy * awv
    w_ref[0, :, 384:512] = fx * fy * awv


def _tc_precompute(query, value, rpx, rpy, Wv, bv2, Wox, box, Woy, boy,
                   Watt, batt2, cf, ci, g, gt):
    bs, len_q, _ = query.shape
    len_v = value.shape[1]
    nqb = len_q // _QBLK
    d = _DMODEL
    c = _NCHAN
    return pl.pallas_call(
        _ab_body,
        grid=(bs, nqb),
        in_specs=[
            pl.BlockSpec((1, _QBLK, d), lambda b, i: (b, i, 0)),
            pl.BlockSpec((1, _QBLK, d), lambda b, i: (b, i, 0)),
            pl.BlockSpec((1, _QBLK, c), lambda b, i: (b, i, 0)),
            pl.BlockSpec((1, _QBLK, c), lambda b, i: (b, i, 0)),
            pl.BlockSpec((d, d), lambda b, i: (0, 0)),
            pl.BlockSpec((1, d), lambda b, i: (0, 0)),
            pl.BlockSpec((d, c), lambda b, i: (0, 0)),
            pl.BlockSpec((1, c), lambda b, i: (0, 0)),
            pl.BlockSpec((d, c), lambda b, i: (0, 0)),
            pl.BlockSpec((1, c), lambda b, i: (0, 0)),
            pl.BlockSpec((d, c), lambda b, i: (0, 0)),
            pl.BlockSpec((1, c), lambda b, i: (0, 0)),
            pl.BlockSpec((4, c), lambda b, i: (0, 0)),
            pl.BlockSpec((3, c), lambda b, i: (0, 0)),
            pl.BlockSpec((c, _NHEADS), lambda b, i: (0, 0)),
            pl.BlockSpec((_NHEADS, c), lambda b, i: (0, 0)),
        ],
        out_specs=[
            pl.BlockSpec((1, _QBLK, d), lambda b, i: (b, i, 0)),
            pl.BlockSpec((1, _QBLK, 4 * c), lambda b, i: (b, i, 0)),
            pl.BlockSpec((1, _QBLK, 4 * c), lambda b, i: (b, i, 0)),
        ],
        out_shape=[
            jax.ShapeDtypeStruct((bs, len_v, d), jnp.float32),
            jax.ShapeDtypeStruct((bs, len_q, 4 * c), jnp.int32),
            jax.ShapeDtypeStruct((bs, len_q, 4 * c), jnp.float32),
        ],
    )(query, value, rpx, rpy, Wv, bv2, Wox, box, Woy, boy, Watt, batt2,
      cf, ci, g, gt)


def _out_body(x_ref, W_ref, b_ref, o_ref):
    o_ref[...] = jnp.dot(x_ref[...], W_ref[...]) + b_ref[...]


def _tc_out(x, Wout, bout2):
    n = x.shape[0]
    d = _DMODEL
    return pl.pallas_call(
        _out_body,
        grid=(n // _QBLK,),
        in_specs=[
            pl.BlockSpec((_QBLK, d), lambda i: (i, 0)),
            pl.BlockSpec((d, d), lambda i: (0, 0)),
            pl.BlockSpec((1, d), lambda i: (0, 0)),
        ],
        out_specs=pl.BlockSpec((_QBLK, d), lambda i: (i, 0)),
        out_shape=jax.ShapeDtypeStruct((n, d), jnp.float32),
    )(x, Wout, bout2)


_NW = 32
_CQ = 4
_JB = _CQ * 4


def _sc_body(table, idxh, wh, outh, idx_v, w_v, rows_v, out_v, sem):
    nq = outh.shape[0]
    per_w = nq // _NW
    wid = lax.axis_index("s") * 2 + lax.axis_index("c")
    q0 = wid * per_w

    def chunk(ci, carry):
        qbase = q0 + ci * _CQ
        r0 = qbase * 4
        pltpu.sync_copy(idxh.at[pl.ds(r0, _JB)], idx_v)
        pltpu.sync_copy(wh.at[pl.ds(r0 * 128, _JB * 128)], w_v)
        cps = [pltpu.async_copy(table.at[idx_v.at[j]], rows_v.at[j], sem)
               for j in range(_JB)]
        for cp in cps:
            cp.wait()

        def group(g, carry2):
            c = g // _NHEADS
            h = g - c * _NHEADS
            acc0 = jnp.zeros((16,), jnp.float32)
            acc1 = jnp.zeros((16,), jnp.float32)
            for corner in range(4):
                blk = c * 4 + corner
                wvec = w_v[pl.ds(blk * 128 + h * 16, 16)]
                for j in range(16):
                    lr = h * 16 + j
                    wj = jnp.full((16,), wvec[j], jnp.float32)
                    acc0 = acc0 + wj * rows_v[blk, lr, 0:16]
                    acc1 = acc1 + wj * rows_v[blk, lr, 16:32]
            out_v[c, pl.ds(h * _HDIM, 16)] = acc0
            out_v[c, pl.ds(h * _HDIM + 16, 16)] = acc1
            return carry2

        lax.fori_loop(0, _CQ * _NHEADS, group, 0)
        pltpu.sync_copy(out_v, outh.at[pl.ds(qbase, _CQ)])
        return carry

    lax.fori_loop(0, per_w // _CQ, chunk, 0)


def _sc_gather(table, idxf, wf, nq):
    mesh = plsc.VectorSubcoreMesh(core_axis_name="c", subcore_axis_name="s")
    fn = pl.kernel(
        _sc_body,
        mesh=mesh,
        compiler_params=pltpu.CompilerParams(use_tc_tiling_on_sc=False),
        out_type=jax.ShapeDtypeStruct((nq, _DMODEL), jnp.float32),
        scratch_types=[
            pltpu.VMEM((_JB, 128), jnp.int32),
            pltpu.VMEM((_JB * 128,), jnp.float32),
            pltpu.VMEM((_JB, 128, _HDIM), jnp.float32),
            pltpu.VMEM((_CQ, _DMODEL), jnp.float32),
            pltpu.SemaphoreType.DMA,
        ],
    )
    return fn(table, idxf, wf)


def kernel(query, value, reference_points, Wv, bv, Woff, boff, Watt, batt,
           Wout, bout):
    bs, len_q, d = query.shape
    len_v = value.shape[1]
    Wo3 = Woff.reshape(d, _NCHAN, 2)
    Wox, Woy = Wo3[:, :, 0], Wo3[:, :, 1]
    bo2 = boff.reshape(_NCHAN, 2)
    box, boy = bo2[:, 0].reshape(1, _NCHAN), bo2[:, 1].reshape(1, _NCHAN)
    bv2 = bv.reshape(1, d)
    batt2 = batt.reshape(1, _NCHAN)
    bout2 = bout.reshape(1, d)
    rp_x = reference_points[..., 0]
    rp_y = reference_points[..., 1]
    rpx = jnp.tile(jnp.repeat(rp_x, _NPOINTS, axis=-1), (1, 1, _NHEADS))
    rpy = jnp.tile(jnp.repeat(rp_y, _NPOINTS, axis=-1), (1, 1, _NHEADS))

    v, idx, wts = _tc_precompute(query, value, rpx, rpy, Wv, bv2, Wox, box,
                                 Woy, boy, Watt, batt2, jnp.asarray(_CF),
                                 jnp.asarray(_CI), jnp.asarray(_GMAT),
                                 jnp.asarray(_GMATT))
    table = v.reshape(bs * len_v * _NHEADS, _HDIM)
    nq = bs * len_q
    idxf = idx.reshape(nq * 4, 128)
    wf = wts.reshape(nq * 4 * 128)
    sampled = _sc_gather(table, idxf, wf, nq)
    out = _tc_out(sampled, Wout, bout2)
    return out.reshape(bs, len_q, d)
